# Initial kernel scaffold; baseline (speedup 1.0000x reference)
import jax
import jax.numpy as jnp
from jax.experimental import pallas as pl


def kernel(left, right):
    raise NotImplementedError("write your pallas kernel here")


# speedup vs baseline: 45.0729x; 45.0729x over previous
import functools

import numpy as np
import jax
import jax.numpy as jnp
from jax import lax
from jax.experimental import pallas as pl
from jax.experimental.pallas import tpu as pltpu
from jax.experimental.pallas import tpu_sc as plsc

_D = 48
_W = 128
_ROW = _W * _D
_LANES = 16
_CHUNKS = _ROW // _LANES


def _build_tables() -> np.ndarray:
    j = np.arange(_ROW)
    w = j // _D
    d = j % _D
    valid = d <= w
    idx_left = np.where(valid, w, _W)
    idx_right = np.where(valid, w - d, _W)
    return np.stack([idx_left, idx_right]).astype(np.int32)


_TABLES = _build_tables()


def kernel(left, right):
    n, c, h, w = left.shape
    assert w == _W
    rows = n * 2 * c * h
    src = jnp.concatenate([left, right], axis=1).reshape(rows, _W)
    tables = jnp.asarray(_TABLES)

    info = plsc.get_sparse_core_info()
    num_cores = info.num_cores
    num_workers = num_cores * info.num_subcores
    rpw = rows // num_workers
    ch_half = c * h

    mesh = plsc.VectorSubcoreMesh(core_axis_name="c", subcore_axis_name="s")

    @functools.partial(
        pl.kernel,
        mesh=mesh,
        compiler_params=pltpu.CompilerParams(needs_layout_passes=False),
        out_type=jax.ShapeDtypeStruct((rows, _ROW), jnp.float32),
        scratch_types=[
            pltpu.VMEM((_W + _LANES,), jnp.float32),
            pltpu.VMEM((_ROW,), jnp.int32),
            pltpu.VMEM((_ROW,), jnp.float32),
            pltpu.VMEM((_ROW,), jnp.float32),
            pltpu.SemaphoreType.DMA,
            pltpu.SemaphoreType.DMA,
        ],
    )
    def sc_body(src_hbm, tab_hbm, out_hbm,
                src_buf, idx_buf, out0, out1, sem0, sem1):
        wid = lax.axis_index("s") * num_cores + lax.axis_index("c")
        base = wid * rpw
        half = (base // ch_half) % 2

        src_buf[pl.ds(_W, _LANES)] = jnp.zeros((_LANES,), jnp.float32)
        pltpu.sync_copy(tab_hbm.at[half], idx_buf)

        def build_row(r, outb):
            pltpu.sync_copy(src_hbm.at[r], src_buf.at[pl.ds(0, _W)])
            for k in range(_CHUNKS):
                idx = idx_buf[pl.ds(_LANES * k, _LANES)]
                outb[pl.ds(_LANES * k, _LANES)] = plsc.load_gather(
                    src_buf, [idx])

        def pair(i, carry):
            for b, (outb, semb) in enumerate(((out0, sem0), (out1, sem1))):
                r = base + 2 * i + b

                @pl.when(i > 0)
                def _drain():
                    pltpu.make_async_copy(outb, out_hbm.at[r], semb).wait()

                build_row(r, outb)
                pltpu.make_async_copy(outb, out_hbm.at[r], semb).start()
            return carry

        lax.fori_loop(0, rpw // 2, pair, 0)
        pltpu.make_async_copy(out0, out_hbm.at[base], sem0).wait()
        pltpu.make_async_copy(out1, out_hbm.at[base], sem1).wait()

    out = sc_body(src, tables)
    return out.reshape(n, 2 * c, h, _W, _D)


# speedup vs baseline: 60.1429x; 1.3343x over previous
import functools

import jax
import jax.numpy as jnp
from jax import lax
from jax.experimental import pallas as pl
from jax.experimental.pallas import tpu as pltpu
from jax.experimental.pallas import tpu_sc as plsc

_D = 48
_W = 128
_ROW = _W * _D
_L = 16
_CHUNKS = _ROW // _L
_B_IN = 32


def kernel(left, right):
    n, c, h, w = left.shape
    assert w == _W
    rows = n * 2 * c * h
    src = jnp.concatenate([left, right], axis=1).reshape(rows, _W)

    info = plsc.get_sparse_core_info()
    num_cores = info.num_cores
    num_workers = num_cores * info.num_subcores
    rpw = rows // num_workers
    ch_half = c * h

    mesh = plsc.VectorSubcoreMesh(core_axis_name="c", subcore_axis_name="s")

    @functools.partial(
        pl.kernel,
        mesh=mesh,
        compiler_params=pltpu.CompilerParams(needs_layout_passes=False),
        out_type=jax.ShapeDtypeStruct((rows, _ROW), jnp.float32),
        scratch_types=[
            pltpu.VMEM((_B_IN, _W), jnp.float32),
            pltpu.VMEM((_ROW,), jnp.float32),
            pltpu.VMEM((_ROW,), jnp.float32),
            pltpu.SemaphoreType.DMA,
            pltpu.SemaphoreType.DMA,
        ],
    )
    def sc_body(src_hbm, out_hbm, src_big, out0, out1, sem0, sem1):
        wid = lax.axis_index("s") * num_cores + lax.axis_index("c")
        base = wid * rpw
        half = (base // ch_half) % 2
        iota = lax.broadcasted_iota(jnp.int32, (_L,), 0)
        zeros = jnp.zeros((_L,), jnp.float32)

        def left_row(br, outb):
            for m in range(8):
                lv = src_big[br, pl.ds(_L * m, _L)]
                for t in range(_L):
                    ww = _L * m + t
                    splat = jnp.full((_L,), lv[t], jnp.float32)
                    for cc in range(3):
                        off = ww * _D + _L * cc
                        if _L * cc + _L - 1 <= ww:
                            outb[pl.ds(off, _L)] = splat
                        elif _L * cc > ww:
                            outb[pl.ds(off, _L)] = zeros
                        else:
                            outb[pl.ds(off, _L)] = jnp.where(
                                iota <= ww - _L * cc, splat, 0.0)

        def right_row(br, outb):
            row_splat = jnp.full((_L,), br, jnp.int32)
            for k in range(_CHUNKS):
                ww = k // 3
                bcol = ww - _L * (k % 3)
                if bcol >= _L - 1:
                    cols = bcol - iota
                    vals = plsc.load_gather(src_big, [row_splat, cols])
                elif bcol < 0:
                    outb[pl.ds(_L * k, _L)] = zeros
                    continue
                else:
                    cols = jnp.maximum(bcol - iota, 0)
                    vals = plsc.load_gather(src_big, [row_splat, cols])
                    vals = jnp.where(iota <= bcol, vals, 0.0)
                outb[pl.ds(_L * k, _L)] = vals

        def run(build_row):
            def pair(i2, carry):
                i = 2 * i2

                @pl.when(lax.rem(i, _B_IN) == 0)
                def _stage():
                    off = pl.multiple_of(base + i, _B_IN)
                    pltpu.sync_copy(src_hbm.at[pl.ds(off, _B_IN)], src_big)

                for b, (outb, semb) in enumerate(((out0, sem0),
                                                  (out1, sem1))):
                    r = base + i + b
                    br = lax.rem(i, _B_IN) + b

                    @pl.when(i2 > 0)
                    def _drain():
                        pltpu.make_async_copy(
                            outb, out_hbm.at[r], semb).wait()

                    build_row(br, outb)
                    pltpu.make_async_copy(outb, out_hbm.at[r], semb).start()
                return carry

            lax.fori_loop(0, rpw // 2, pair, 0)
            pltpu.make_async_copy(out0, out_hbm.at[base], sem0).wait()
            pltpu.make_async_copy(out1, out_hbm.at[base], sem1).wait()

        @pl.when(half == 0)
        def _left():
            run(left_row)

        @pl.when(half == 1)
        def _right():
            run(right_row)

    out = sc_body(src)
    return out.reshape(n, 2 * c, h, _W, _D)


# speedup vs baseline: 98.4729x; 1.6373x over previous
import functools

import jax
import jax.numpy as jnp
from jax import lax
from jax.experimental import pallas as pl
from jax.experimental.pallas import tpu as pltpu
from jax.experimental.pallas import tpu_sc as plsc

_D = 48
_W = 128
_L = 16
_B_IN = 32


def kernel(left, right):
    n, c, h, w = left.shape
    assert w == _W
    rows = n * 2 * c * h
    src = jnp.concatenate([left, right], axis=1).reshape(rows, _W)

    info = plsc.get_sparse_core_info()
    num_cores = info.num_cores
    num_workers = num_cores * info.num_subcores
    rpw = rows // num_workers
    ch_half = c * h

    mesh = plsc.VectorSubcoreMesh(core_axis_name="c", subcore_axis_name="s")

    @functools.partial(
        pl.kernel,
        mesh=mesh,
        compiler_params=pltpu.CompilerParams(needs_layout_passes=False),
        out_type=jax.ShapeDtypeStruct((rows, _W, _D), jnp.float32),
        scratch_types=[
            pltpu.VMEM((_B_IN, _W), jnp.float32),
            pltpu.VMEM((_W, _D), jnp.float32),
            pltpu.VMEM((_W, _D), jnp.float32),
            pltpu.SemaphoreType.DMA,
            pltpu.SemaphoreType.DMA,
        ],
    )
    def sc_body(src_hbm, out_hbm, src_big, out0, out1, sem0, sem1):
        wid = lax.axis_index("s") * num_cores + lax.axis_index("c")
        base = wid * rpw
        half = (base // ch_half) % 2
        iota = lax.broadcasted_iota(jnp.int32, (_L,), 0)
        zeros = jnp.zeros((_L,), jnp.float32)

        def left_row(br, outb):
            for m in range(8):
                lv = src_big[br, pl.ds(_L * m, _L)]
                for t in range(_L):
                    ww = _L * m + t
                    splat = jnp.full((_L,), lv[t], jnp.float32)
                    for cc in range(3):
                        if _L * cc + _L - 1 <= ww:
                            outb[ww, pl.ds(_L * cc, _L)] = splat
                        elif _L * cc > ww:
                            outb[ww, pl.ds(_L * cc, _L)] = zeros
                        else:
                            outb[ww, pl.ds(_L * cc, _L)] = jnp.where(
                                iota <= ww - _L * cc, splat, 0.0)

        def right_row(br, outb):
            row_splat = jnp.full((_L,), br, jnp.int32)
            for ww in range(_W):
                for cc in range(3):
                    bcol = ww - _L * cc
                    if bcol >= _L - 1:
                        cols = bcol - iota
                        vals = plsc.load_gather(src_big, [row_splat, cols])
                    elif bcol < 0:
                        outb[ww, pl.ds(_L * cc, _L)] = zeros
                        continue
                    else:
                        cols = jnp.maximum(bcol - iota, 0)
                        vals = plsc.load_gather(src_big, [row_splat, cols])
                        vals = jnp.where(iota <= bcol, vals, 0.0)
                    outb[ww, pl.ds(_L * cc, _L)] = vals

        def run(build_row):
            def pair(i2, carry):
                i = 2 * i2

                @pl.when(lax.rem(i, _B_IN) == 0)
                def _stage():
                    off = pl.multiple_of(base + i, _B_IN)
                    pltpu.sync_copy(src_hbm.at[pl.ds(off, _B_IN)], src_big)

                for b, (outb, semb) in enumerate(((out0, sem0),
                                                  (out1, sem1))):
                    r = base + i + b
                    br = lax.rem(i, _B_IN) + b

                    @pl.when(i2 > 0)
                    def _drain():
                        pltpu.make_async_copy(
                            outb, out_hbm.at[r], semb).wait()

                    build_row(br, outb)
                    pltpu.make_async_copy(outb, out_hbm.at[r], semb).start()
                return carry

            lax.fori_loop(0, rpw // 2, pair, 0)
            pltpu.make_async_copy(out0, out_hbm.at[base], sem0).wait()
            pltpu.make_async_copy(out1, out_hbm.at[base], sem1).wait()

        @pl.when(half == 0)
        def _left():
            run(left_row)

        @pl.when(half == 1)
        def _right():
            run(right_row)

    out = sc_body(src)
    return out.reshape(n, 2 * c, h, _W, _D)


# speedup vs baseline: 190.4112x; 1.9336x over previous
import functools

import jax
import jax.numpy as jnp
from jax import lax
from jax.experimental import pallas as pl
from jax.experimental.pallas import tpu as pltpu
from jax.experimental.pallas import tpu_sc as plsc

_D = 48
_W = 128
_L = 16
_NCH = _W // _L
_B_IN = 32
_PAD = 128


def kernel(left, right):
    n, c, h, w = left.shape
    assert w == _W
    rows = n * 2 * c * h
    src = jnp.concatenate([left, right], axis=1).reshape(rows, _W)

    info = plsc.get_sparse_core_info()
    num_cores = info.num_cores
    num_workers = num_cores * info.num_subcores
    rpw = rows // num_workers
    ch_half = c * h

    mesh = plsc.VectorSubcoreMesh(core_axis_name="c", subcore_axis_name="s")

    @functools.partial(
        pl.kernel,
        mesh=mesh,
        compiler_params=pltpu.CompilerParams(needs_layout_passes=False),
        out_type=jax.ShapeDtypeStruct((rows, _D, _W), jnp.float32),
        scratch_types=[
            pltpu.VMEM((_B_IN, _PAD + _W), jnp.float32),
            pltpu.VMEM((_D, _W), jnp.float32),
            pltpu.VMEM((_D, _W), jnp.float32),
            pltpu.SemaphoreType.DMA,
            pltpu.SemaphoreType.DMA,
        ],
    )
    def sc_body(src_hbm, out_hbm, src_big, out0, out1, sem0, sem1):
        wid = lax.axis_index("s") * num_cores + lax.axis_index("c")
        base = wid * rpw
        half = (base // ch_half) % 2
        iota = lax.broadcasted_iota(jnp.int32, (_L,), 0)
        zeros = jnp.zeros((_L,), jnp.float32)

        for rr in range(_B_IN):
            for mm in range(_PAD // _L):
                src_big[rr, pl.ds(_L * mm, _L)] = zeros

        def left_row(br, outb):
            lv = [src_big[br, pl.ds(_PAD + _L * m, _L)] for m in range(_NCH)]
            for d in range(_D):
                mp = d // _L
                for m in range(_NCH):
                    if m < mp:
                        outb[d, pl.ds(_L * m, _L)] = zeros
                    elif m == mp and d % _L != 0:
                        outb[d, pl.ds(_L * m, _L)] = jnp.where(
                            iota >= d - _L * m, lv[m], 0.0)
                    else:
                        outb[d, pl.ds(_L * m, _L)] = lv[m]

        def right_row(br, outb):
            for d in range(_D):
                mp = d // _L
                for m in range(_NCH):
                    if m < mp:
                        outb[d, pl.ds(_L * m, _L)] = zeros
                    else:
                        outb[d, pl.ds(_L * m, _L)] = src_big[
                            br, pl.ds(_PAD + _L * m - d, _L)]

        def run(build_row):
            def pair(i2, carry):
                i = 2 * i2

                @pl.when(lax.rem(i, _B_IN) == 0)
                def _stage():
                    off = pl.multiple_of(base + i, _B_IN)
                    pltpu.sync_copy(src_hbm.at[pl.ds(off, _B_IN)],
                                    src_big.at[:, pl.ds(_PAD, _W)])

                for b, (outb, semb) in enumerate(((out0, sem0),
                                                  (out1, sem1))):
                    r = base + i + b
                    br = lax.rem(i, _B_IN) + b

                    @pl.when(i2 > 0)
                    def _drain():
                        pltpu.make_async_copy(
                            outb, out_hbm.at[r], semb).wait()

                    build_row(br, outb)
                    pltpu.make_async_copy(outb, out_hbm.at[r], semb).start()
                return carry

            lax.fori_loop(0, rpw // 2, pair, 0)
            pltpu.make_async_copy(out0, out_hbm.at[base], sem0).wait()
            pltpu.make_async_copy(out1, out_hbm.at[base], sem1).wait()

        @pl.when(half == 0)
        def _left():
            run(left_row)

        @pl.when(half == 1)
        def _right():
            run(right_row)

    out = sc_body(src)
    return jnp.swapaxes(out.reshape(n, 2 * c, h, _D, _W), -1, -2)
